# baseline (speedup 1.0000x reference)
import jax
import jax.numpy as jnp
from jax import lax
from jax.experimental import pallas as pl
from jax.experimental.pallas import tpu as pltpu
from jax.experimental.pallas import tpu_sc as plsc

_EPS = 1e-5
_NC = 2
_NS = 16
_BLK = 128
_CHUNKS = (1000, 750, 500, 250)
_NCHUNK = len(_CHUNKS)


def _ln_rows(v, g, b):
    mu = jnp.mean(v, axis=-1, keepdims=True)
    dv = v - mu
    var = jnp.mean(dv * dv, axis=-1, keepdims=True)
    return g * dv * lax.rsqrt(var + _EPS) + b


def _dot_t(a, w):
    return lax.dot_general(a, w, (((1,), (1,)), ((), ())),
                           preferred_element_type=jnp.float32)


def _qk_body(x_ref, wq_ref, bq_ref, wk_ref, bk_ref, q_ref, k_ref):
    xb = x_ref[...]
    q_ref[...] = _dot_t(xb, wq_ref[...]) + bq_ref[...]
    k_ref[...] = _dot_t(xb, wk_ref[...]) + bk_ref[...]


def _edge_body(pc_ref, g_ref, we_ref, wc_ref, bc_ref, ge_ref, be_ref,
               conn_ref, e_ref):
    pc = pc_ref[...]
    eh = _dot_t(pc, we_ref[...])
    c2 = jnp.maximum(g_ref[...] + eh, 0.0)
    conn = _dot_t(c2, wc_ref[...]) + bc_ref[...]
    conn_ref[...] = conn
    e_ref[...] = _ln_rows(pc + conn, ge_ref[...], be_ref[...])


def _edge_body_alias(pc_ref, g_ref, we_ref, wc_ref, bc_ref, ge_ref, be_ref,
                     ebuf_ref, conn_ref, e_ref):
    del ebuf_ref
    _edge_body(pc_ref, g_ref, we_ref, wc_ref, bc_ref, ge_ref, be_ref,
               conn_ref, e_ref)


def _final_body(*refs):
    p_refs = refs[:_NCHUNK]
    x_ref, wn_ref, bn_ref, gh_ref, bh_ref, h_ref = refs[_NCHUNK:]
    agg = p_refs[0][0] + p_refs[0][1]
    for p in p_refs[1:]:
        agg = agg + (p[0] + p[1])
    hb = _dot_t(agg, wn_ref[...]) + bn_ref[...]
    h_ref[...] = _ln_rows(x_ref[...] + hb, gh_ref[...], bh_ref[...])


def _build_gather(d, blk_lo, nblk_c):
    nw = _NC * _NS
    t = max(-(-nblk_c // nw), 3)
    groups = t // 3
    rem = t % 3
    mesh = plsc.VectorSubcoreMesh(core_axis_name="c", subcore_axis_name="s",
                                  num_cores=_NC, num_subcores=_NS)

    def body(q_hbm, k_hbm, dst_hbm, src_hbm, g_hbm,
             idxd, idxs,
             rq0, rq1, rq2, rk0, rk1, rk2,
             sg0, sg1, sg2, sw0, sw1, sw2):
        rqs = (rq0, rq1, rq2)
        rks = (rk0, rk1, rk2)
        sgs = (sg0, sg1, sg2)
        sws = (sw0, sw1, sw2)
        c = lax.axis_index("c")
        s = lax.axis_index("s")
        w = s * _NC + c
        start = (w * (nblk_c - t)) // (nw - 1)

        goff = (blk_lo + start) * _BLK
        pltpu.sync_copy(dst_hbm.at[pl.ds(goff, t * _BLK)], idxd)
        pltpu.sync_copy(src_hbm.at[pl.ds(goff, t * _BLK)], idxs)

        def gather_into(bl, b):
            di = idxd.at[pl.ds(bl * _BLK, _BLK)]
            si = idxs.at[pl.ds(bl * _BLK, _BLK)]
            pltpu.async_copy(q_hbm.at[di], rqs[b], sgs[b])
            pltpu.async_copy(k_hbm.at[si], rks[b], sgs[b])

        def wait_gather(bl, b):
            di = idxd.at[pl.ds(bl * _BLK, _BLK)]
            si = idxs.at[pl.ds(bl * _BLK, _BLK)]
            pltpu.make_async_copy(q_hbm.at[di], rqs[b], sgs[b]).wait()
            pltpu.make_async_copy(k_hbm.at[si], rks[b], sgs[b]).wait()

        def wb(bl, b):
            o = pl.ds((start + bl) * _BLK, _BLK)
            pltpu.async_copy(rqs[b], g_hbm.at[o], sws[b])

        def wait_wb(bl, b):
            o = pl.ds((start + bl) * _BLK, _BLK)
            pltpu.make_async_copy(rqs[b], g_hbm.at[o], sws[b]).wait()

        def do_add(b):
            rq, rk = rqs[b], rks[b]

            def row(r, carry2):
                for j in range(d // 16):
                    sl = pl.ds(j * 16, 16)
                    rq[r, sl] = rq[r, sl] + rk[r, sl]
                return carry2

            lax.fori_loop(0, _BLK, row, 0)

        gather_into(0, 0)
        gather_into(1, 1)

        def group(g, carry):
            for b in range(3):
                bl = g * 3 + b
                wait_gather(bl, b)
                do_add(b)
                wb(bl, b)
                nb = (b + 2) % 3
                if b == 0:
                    @pl.when(g > 0)
                    def _():
                        wait_wb(bl - 1, nb)
                        gather_into(bl + 2, nb)

                    @pl.when(g == 0)
                    def _():
                        gather_into(bl + 2, nb)
                else:
                    @pl.when(bl + 2 < t)
                    def _():
                        wait_wb(bl - 1, nb)
                        gather_into(bl + 2, nb)
            return carry

        lax.fori_loop(0, groups, group, 0)
        for bl in range(t - rem, t):
            b = bl % 3
            wait_gather(bl, b)
            do_add(b)
            wb(bl, b)
        for bl in range(t - 3, t):
            wait_wb(bl, bl % 3)

    return pl.kernel(
        body,
        out_type=jax.ShapeDtypeStruct((nblk_c * _BLK, d), jnp.float32),
        mesh=mesh,
        scratch_types=[
            pltpu.VMEM((t * _BLK,), jnp.int32),
            pltpu.VMEM((t * _BLK,), jnp.int32),
        ] + [pltpu.VMEM((_BLK, d), jnp.float32)] * 6
          + [pltpu.SemaphoreType.DMA] * 6,
    )


def _build_segsum(n, d, blk_lo, nblk_c):
    nw = _NC * _NS
    pairs = nblk_c // 2
    pbase, prem = divmod(pairs, nw)
    npad = -(-n // (8 * _NS)) * (8 * _NS)
    rpt = npad // _NS
    mesh = plsc.VectorSubcoreMesh(core_axis_name="c", subcore_axis_name="s",
                                  num_cores=_NC, num_subcores=_NS)

    chunks = [(co, _BLK) for co in range(0, rpt - rpt % _BLK, _BLK)]
    if rpt % _BLK:
        chunks.append((rpt - rpt % _BLK, rpt % _BLK))

    def body(conn_hbm, dst_hbm, out_hbm, idx0, idx1, rows0, rows1, acc,
             sc0, sc1):
        idxs = (idx0, idx1)
        rows = (rows0, rows1)
        scs = (sc0, sc1)
        c = lax.axis_index("c")
        s = lax.axis_index("s")
        w = s * _NC + c

        def zrow(r, carry):
            for j in range(d // 16):
                rows0[r, pl.ds(j * 16, 16)] = jnp.zeros((16,), jnp.float32)
            return carry

        lax.fori_loop(0, _BLK, zrow, 0)
        ro = s * rpt
        for co, cl in chunks:
            pltpu.sync_copy(rows0.at[pl.ds(0, cl)], acc.at[pl.ds(ro + co, cl)])
        plsc.subcore_barrier()

        start = (w * pbase + jnp.minimum(w, prem)) * 2
        cntb = (pbase + jnp.where(w < prem, 1, 0)) * 2

        def load_blk(k, b):
            pltpu.async_copy(dst_hbm.at[pl.ds((blk_lo + k) * _BLK, _BLK)],
                             idxs[b], scs[b])
            pltpu.async_copy(conn_hbm.at[pl.ds(k * _BLK, _BLK)],
                             rows[b], scs[b])

        def wait_blk(k, b):
            pltpu.make_async_copy(dst_hbm.at[pl.ds((blk_lo + k) * _BLK, _BLK)],
                                  idxs[b], scs[b]).wait()
            pltpu.make_async_copy(conn_hbm.at[pl.ds(k * _BLK, _BLK)],
                                  rows[b], scs[b]).wait()

        load_blk(start, 0)
        load_blk(start + 1, 1)

        def pair_body(p, carry):
            for b in range(2):
                k = start + p * 2 + b
                wait_blk(k, b)
                pltpu.sync_copy(rows[b], acc.at[idxs[b]], add=True)

                @pl.when(p * 2 + b + 2 < cntb)
                def _():
                    load_blk(k + 2, b)
            return carry

        lax.fori_loop(0, cntb // 2, pair_body, 0)
        plsc.subcore_barrier()
        for co, cl in chunks:
            pltpu.sync_copy(acc.at[pl.ds(ro + co, cl)],
                            rows0.at[pl.ds(0, cl)])
            pltpu.sync_copy(rows0.at[pl.ds(0, cl)],
                            out_hbm.at[c, pl.ds(ro + co, cl)])

    return pl.kernel(
        body,
        out_type=jax.ShapeDtypeStruct((_NC, npad, d), jnp.float32),
        mesh=mesh,
        scratch_types=[
            pltpu.VMEM((_BLK,), jnp.int32),
            pltpu.VMEM((_BLK,), jnp.int32),
            pltpu.VMEM((_BLK, d), jnp.float32),
            pltpu.VMEM((_BLK, d), jnp.float32),
            pltpu.VMEM_SHARED((npad, d), jnp.float32),
            pltpu.SemaphoreType.DMA,
            pltpu.SemaphoreType.DMA,
        ],
    )


def kernel(x, poly_conn, poly_index, WQ, bQ, WK, bK, WE, Wc, bc, Wn, bn,
           g_h, b_h, g_e, b_e):
    n, d = x.shape
    e = poly_conn.shape[0]
    dst = poly_index[0]
    src = poly_index[1]

    bnr = 2000
    ber = 2000
    clos = [sum(_CHUNKS[:i]) for i in range(_NCHUNK)]
    npad = -(-n // (8 * _NS)) * (8 * _NS)

    w_spec = pl.BlockSpec((d, d), lambda i: (0, 0))
    v_spec = pl.BlockSpec((1, d), lambda i: (0, 0))
    nb_spec = pl.BlockSpec((bnr, d), lambda i: (i, 0))

    qall, kall = pl.pallas_call(
        _qk_body,
        grid=(n // bnr,),
        in_specs=[nb_spec, w_spec, v_spec, w_spec, v_spec],
        out_specs=[nb_spec, nb_spec],
        out_shape=[jax.ShapeDtypeStruct((n, d), jnp.float32)] * 2,
    )(x, WQ, bQ.reshape(1, d), WK, bK.reshape(1, d))

    gs = [_build_gather(d, clos[ci], _CHUNKS[ci])(qall, kall, dst, src)
          for ci in range(_NCHUNK)]

    conns = []
    e_buf = None
    for ci in range(_NCHUNK):
        ec = _CHUNKS[ci] * _BLK
        eb_lo = clos[ci] * _BLK // ber
        in_specs = [
            pl.BlockSpec((ber, d), lambda i, o=eb_lo: (i + o, 0)),
            pl.BlockSpec((ber, d), lambda i: (i, 0)),
            w_spec, w_spec, v_spec, v_spec, v_spec,
        ]
        args = [poly_conn, gs[ci], WE, Wc, bc.reshape(1, d),
                g_e.reshape(1, d), b_e.reshape(1, d)]
        if ci == 0:
            body, aliases = _edge_body, {}
        else:
            in_specs.append(pl.BlockSpec(memory_space=pl.ANY))
            args.append(e_buf)
            body, aliases = _edge_body_alias, {7: 1}
        conn_c, e_buf = pl.pallas_call(
            body,
            grid=(ec // ber,),
            in_specs=in_specs,
            out_specs=[pl.BlockSpec((ber, d), lambda i: (i, 0)),
                       pl.BlockSpec((ber, d), lambda i, o=eb_lo: (i + o, 0))],
            out_shape=[jax.ShapeDtypeStruct((ec, d), jnp.float32),
                       jax.ShapeDtypeStruct((e, d), jnp.float32)],
            input_output_aliases=aliases,
        )(*args)
        conns.append(conn_c)

    parts = [_build_segsum(n, d, clos[ci], _CHUNKS[ci])(conns[ci], dst)
             for ci in range(_NCHUNK)]

    h = pl.pallas_call(
        _final_body,
        grid=(n // bnr,),
        in_specs=[pl.BlockSpec((_NC, bnr, d), lambda i: (0, i, 0))] * _NCHUNK
                 + [nb_spec, w_spec, v_spec, v_spec, v_spec],
        out_specs=nb_spec,
        out_shape=jax.ShapeDtypeStruct((n, d), jnp.float32),
    )(*parts, x, Wn, bn.reshape(1, d), g_h.reshape(1, d), b_h.reshape(1, d))

    return h, e_buf


# speedup vs baseline: 1.0165x; 1.0165x over previous
import jax
import jax.numpy as jnp
from jax import lax
from jax.experimental import pallas as pl
from jax.experimental.pallas import tpu as pltpu
from jax.experimental.pallas import tpu_sc as plsc

_EPS = 1e-5
_NC = 2
_NS = 16
_BLK = 128
_CHUNKS = (1250, 750, 500)
_NCHUNK = len(_CHUNKS)


def _ln_rows(v, g, b):
    mu = jnp.mean(v, axis=-1, keepdims=True)
    dv = v - mu
    var = jnp.mean(dv * dv, axis=-1, keepdims=True)
    return g * dv * lax.rsqrt(var + _EPS) + b


def _dot_t(a, w):
    return lax.dot_general(a, w, (((1,), (1,)), ((), ())),
                           preferred_element_type=jnp.float32)


def _qk_body(x_ref, wq_ref, bq_ref, wk_ref, bk_ref, q_ref, k_ref):
    xb = x_ref[...]
    q_ref[...] = _dot_t(xb, wq_ref[...]) + bq_ref[...]
    k_ref[...] = _dot_t(xb, wk_ref[...]) + bk_ref[...]


def _edge_body(pc_ref, g_ref, we_ref, wc_ref, bc_ref, ge_ref, be_ref,
               conn_ref, e_ref):
    pc = pc_ref[...]
    eh = _dot_t(pc, we_ref[...])
    c2 = jnp.maximum(g_ref[...] + eh, 0.0)
    conn = _dot_t(c2, wc_ref[...]) + bc_ref[...]
    conn_ref[...] = conn
    e_ref[...] = _ln_rows(pc + conn, ge_ref[...], be_ref[...])


def _edge_body_alias(pc_ref, g_ref, we_ref, wc_ref, bc_ref, ge_ref, be_ref,
                     ebuf_ref, conn_ref, e_ref):
    del ebuf_ref
    _edge_body(pc_ref, g_ref, we_ref, wc_ref, bc_ref, ge_ref, be_ref,
               conn_ref, e_ref)


def _final_body(*refs):
    p_refs = refs[:_NCHUNK]
    x_ref, wn_ref, bn_ref, gh_ref, bh_ref, h_ref = refs[_NCHUNK:]
    agg = p_refs[0][0] + p_refs[0][1]
    for p in p_refs[1:]:
        agg = agg + (p[0] + p[1])
    hb = _dot_t(agg, wn_ref[...]) + bn_ref[...]
    h_ref[...] = _ln_rows(x_ref[...] + hb, gh_ref[...], bh_ref[...])


def _build_gather(d, blk_lo, nblk_c):
    nw = _NC * _NS
    t = max(-(-nblk_c // nw), 3)
    groups = t // 3
    rem = t % 3
    mesh = plsc.VectorSubcoreMesh(core_axis_name="c", subcore_axis_name="s",
                                  num_cores=_NC, num_subcores=_NS)

    def body(q_hbm, k_hbm, dst_hbm, src_hbm, g_hbm,
             idxd, idxs,
             rq0, rq1, rq2, rk0, rk1, rk2,
             sg0, sg1, sg2, sw0, sw1, sw2):
        rqs = (rq0, rq1, rq2)
        rks = (rk0, rk1, rk2)
        sgs = (sg0, sg1, sg2)
        sws = (sw0, sw1, sw2)
        c = lax.axis_index("c")
        s = lax.axis_index("s")
        w = s * _NC + c
        start = (w * (nblk_c - t)) // (nw - 1)

        goff = (blk_lo + start) * _BLK
        pltpu.sync_copy(dst_hbm.at[pl.ds(goff, t * _BLK)], idxd)
        pltpu.sync_copy(src_hbm.at[pl.ds(goff, t * _BLK)], idxs)

        def gather_into(bl, b):
            di = idxd.at[pl.ds(bl * _BLK, _BLK)]
            si = idxs.at[pl.ds(bl * _BLK, _BLK)]
            pltpu.async_copy(q_hbm.at[di], rqs[b], sgs[b])
            pltpu.async_copy(k_hbm.at[si], rks[b], sgs[b])

        def wait_gather(bl, b):
            di = idxd.at[pl.ds(bl * _BLK, _BLK)]
            si = idxs.at[pl.ds(bl * _BLK, _BLK)]
            pltpu.make_async_copy(q_hbm.at[di], rqs[b], sgs[b]).wait()
            pltpu.make_async_copy(k_hbm.at[si], rks[b], sgs[b]).wait()

        def wb(bl, b):
            o = pl.ds((start + bl) * _BLK, _BLK)
            pltpu.async_copy(rqs[b], g_hbm.at[o], sws[b])

        def wait_wb(bl, b):
            o = pl.ds((start + bl) * _BLK, _BLK)
            pltpu.make_async_copy(rqs[b], g_hbm.at[o], sws[b]).wait()

        def do_add(b):
            rq, rk = rqs[b], rks[b]

            def row(r, carry2):
                for j in range(d // 16):
                    sl = pl.ds(j * 16, 16)
                    rq[r, sl] = rq[r, sl] + rk[r, sl]
                return carry2

            lax.fori_loop(0, _BLK, row, 0)

        gather_into(0, 0)
        gather_into(1, 1)

        def group(g, carry):
            for b in range(3):
                bl = g * 3 + b
                wait_gather(bl, b)
                do_add(b)
                wb(bl, b)
                nb = (b + 2) % 3
                if b == 0:
                    @pl.when(g > 0)
                    def _():
                        wait_wb(bl - 1, nb)
                        gather_into(bl + 2, nb)

                    @pl.when(g == 0)
                    def _():
                        gather_into(bl + 2, nb)
                else:
                    @pl.when(bl + 2 < t)
                    def _():
                        wait_wb(bl - 1, nb)
                        gather_into(bl + 2, nb)
            return carry

        lax.fori_loop(0, groups, group, 0)
        for bl in range(t - rem, t):
            b = bl % 3
            wait_gather(bl, b)
            do_add(b)
            wb(bl, b)
        for bl in range(t - 3, t):
            wait_wb(bl, bl % 3)

    return pl.kernel(
        body,
        out_type=jax.ShapeDtypeStruct((nblk_c * _BLK, d), jnp.float32),
        mesh=mesh,
        scratch_types=[
            pltpu.VMEM((t * _BLK,), jnp.int32),
            pltpu.VMEM((t * _BLK,), jnp.int32),
        ] + [pltpu.VMEM((_BLK, d), jnp.float32)] * 6
          + [pltpu.SemaphoreType.DMA] * 6,
    )


def _build_segsum(n, d, blk_lo, nblk_c):
    nw = _NC * _NS
    pairs = nblk_c // 2
    pbase, prem = divmod(pairs, nw)
    npad = -(-n // (8 * _NS)) * (8 * _NS)
    rpt = npad // _NS
    mesh = plsc.VectorSubcoreMesh(core_axis_name="c", subcore_axis_name="s",
                                  num_cores=_NC, num_subcores=_NS)

    chunks = [(co, _BLK) for co in range(0, rpt - rpt % _BLK, _BLK)]
    if rpt % _BLK:
        chunks.append((rpt - rpt % _BLK, rpt % _BLK))

    def body(conn_hbm, dst_hbm, out_hbm, idx0, idx1, rows0, rows1, acc,
             sc0, sc1):
        idxs = (idx0, idx1)
        rows = (rows0, rows1)
        scs = (sc0, sc1)
        c = lax.axis_index("c")
        s = lax.axis_index("s")
        w = s * _NC + c

        def zrow(r, carry):
            for j in range(d // 16):
                rows0[r, pl.ds(j * 16, 16)] = jnp.zeros((16,), jnp.float32)
            return carry

        lax.fori_loop(0, _BLK, zrow, 0)
        ro = s * rpt
        for co, cl in chunks:
            pltpu.sync_copy(rows0.at[pl.ds(0, cl)], acc.at[pl.ds(ro + co, cl)])
        plsc.subcore_barrier()

        start = (w * pbase + jnp.minimum(w, prem)) * 2
        cntb = (pbase + jnp.where(w < prem, 1, 0)) * 2

        def load_blk(k, b):
            pltpu.async_copy(dst_hbm.at[pl.ds((blk_lo + k) * _BLK, _BLK)],
                             idxs[b], scs[b])
            pltpu.async_copy(conn_hbm.at[pl.ds(k * _BLK, _BLK)],
                             rows[b], scs[b])

        def wait_blk(k, b):
            pltpu.make_async_copy(dst_hbm.at[pl.ds((blk_lo + k) * _BLK, _BLK)],
                                  idxs[b], scs[b]).wait()
            pltpu.make_async_copy(conn_hbm.at[pl.ds(k * _BLK, _BLK)],
                                  rows[b], scs[b]).wait()

        load_blk(start, 0)
        load_blk(start + 1, 1)

        def pair_body(p, carry):
            for b in range(2):
                k = start + p * 2 + b
                wait_blk(k, b)
                pltpu.sync_copy(rows[b], acc.at[idxs[b]], add=True)

                @pl.when(p * 2 + b + 2 < cntb)
                def _():
                    load_blk(k + 2, b)
            return carry

        lax.fori_loop(0, cntb // 2, pair_body, 0)
        plsc.subcore_barrier()
        for co, cl in chunks:
            pltpu.sync_copy(acc.at[pl.ds(ro + co, cl)],
                            rows0.at[pl.ds(0, cl)])
            pltpu.sync_copy(rows0.at[pl.ds(0, cl)],
                            out_hbm.at[c, pl.ds(ro + co, cl)])

    return pl.kernel(
        body,
        out_type=jax.ShapeDtypeStruct((_NC, npad, d), jnp.float32),
        mesh=mesh,
        scratch_types=[
            pltpu.VMEM((_BLK,), jnp.int32),
            pltpu.VMEM((_BLK,), jnp.int32),
            pltpu.VMEM((_BLK, d), jnp.float32),
            pltpu.VMEM((_BLK, d), jnp.float32),
            pltpu.VMEM_SHARED((npad, d), jnp.float32),
            pltpu.SemaphoreType.DMA,
            pltpu.SemaphoreType.DMA,
        ],
    )


def kernel(x, poly_conn, poly_index, WQ, bQ, WK, bK, WE, Wc, bc, Wn, bn,
           g_h, b_h, g_e, b_e):
    n, d = x.shape
    e = poly_conn.shape[0]
    dst = poly_index[0]
    src = poly_index[1]

    bnr = 2000
    ber = 2000
    clos = [sum(_CHUNKS[:i]) for i in range(_NCHUNK)]
    npad = -(-n // (8 * _NS)) * (8 * _NS)

    w_spec = pl.BlockSpec((d, d), lambda i: (0, 0))
    v_spec = pl.BlockSpec((1, d), lambda i: (0, 0))
    nb_spec = pl.BlockSpec((bnr, d), lambda i: (i, 0))

    qall, kall = pl.pallas_call(
        _qk_body,
        grid=(n // bnr,),
        in_specs=[nb_spec, w_spec, v_spec, w_spec, v_spec],
        out_specs=[nb_spec, nb_spec],
        out_shape=[jax.ShapeDtypeStruct((n, d), jnp.float32)] * 2,
    )(x, WQ, bQ.reshape(1, d), WK, bK.reshape(1, d))

    gs = [_build_gather(d, clos[ci], _CHUNKS[ci])(qall, kall, dst, src)
          for ci in range(_NCHUNK)]

    conns = []
    e_buf = None
    for ci in range(_NCHUNK):
        ec = _CHUNKS[ci] * _BLK
        eb_lo = clos[ci] * _BLK // ber
        in_specs = [
            pl.BlockSpec((ber, d), lambda i, o=eb_lo: (i + o, 0)),
            pl.BlockSpec((ber, d), lambda i: (i, 0)),
            w_spec, w_spec, v_spec, v_spec, v_spec,
        ]
        args = [poly_conn, gs[ci], WE, Wc, bc.reshape(1, d),
                g_e.reshape(1, d), b_e.reshape(1, d)]
        if ci == 0:
            body, aliases = _edge_body, {}
        else:
            in_specs.append(pl.BlockSpec(memory_space=pl.ANY))
            args.append(e_buf)
            body, aliases = _edge_body_alias, {7: 1}
        conn_c, e_buf = pl.pallas_call(
            body,
            grid=(ec // ber,),
            in_specs=in_specs,
            out_specs=[pl.BlockSpec((ber, d), lambda i: (i, 0)),
                       pl.BlockSpec((ber, d), lambda i, o=eb_lo: (i + o, 0))],
            out_shape=[jax.ShapeDtypeStruct((ec, d), jnp.float32),
                       jax.ShapeDtypeStruct((e, d), jnp.float32)],
            input_output_aliases=aliases,
        )(*args)
        conns.append(conn_c)

    parts = [_build_segsum(n, d, clos[ci], _CHUNKS[ci])(conns[ci], dst)
             for ci in range(_NCHUNK)]

    h = pl.pallas_call(
        _final_body,
        grid=(n // bnr,),
        in_specs=[pl.BlockSpec((_NC, bnr, d), lambda i: (0, i, 0))] * _NCHUNK
                 + [nb_spec, w_spec, v_spec, v_spec, v_spec],
        out_specs=nb_spec,
        out_shape=jax.ShapeDtypeStruct((n, d), jnp.float32),
    )(*parts, x, Wn, bn.reshape(1, d), g_h.reshape(1, d), b_h.reshape(1, d))

    return h, e_buf
